# initial kernel scaffold (unmeasured)
import jax
import jax.numpy as jnp
from jax import lax
from jax.experimental import pallas as pl
from jax.experimental.pallas import tpu as pltpu

N_DEV = 4


def kernel(x, w_mat):
    m, k_per = x.shape
    _, n = w_mat.shape
    m_chunk = m // N_DEV
    n_hops = 2 * (N_DEV - 1)

    def body(x_ref, w_ref, out_ref, w_bf_ref, comm_ref, send_sems, recv_sems):
        my = lax.axis_index("i")
        left = lax.rem(my + (N_DEV - 1), N_DEV)
        right = lax.rem(my + 1, N_DEV)

        barrier_sem = pltpu.get_barrier_semaphore()
        for nbr in (left, right):
            pl.semaphore_signal(
                barrier_sem, inc=1,
                device_id=(nbr,), device_id_type=pl.DeviceIdType.MESH,
            )
        pl.semaphore_wait(barrier_sem, 2)

        w_bf_ref[...] = w_ref[...].astype(jnp.bfloat16)

        def partial_chunk(c):
            xs = x_ref[pl.ds(c * m_chunk, m_chunk), :].astype(jnp.bfloat16)
            return jnp.dot(xs, w_bf_ref[...], preferred_element_type=jnp.float32)

        def hop(h, to_send_slot=None):
            send_slot = h % 2
            recv_slot = (h + 1) % 2
            if to_send_slot is not None:
                comm_ref[send_slot, :, :] = to_send_slot
            rdma = pltpu.make_async_remote_copy(
                src_ref=comm_ref.at[send_slot],
                dst_ref=comm_ref.at[recv_slot],
                send_sem=send_sems.at[h],
                recv_sem=recv_sems.at[h],
                device_id=(right,),
                device_id_type=pl.DeviceIdType.MESH,
            )
            rdma.start()
            rdma.wait()
            return recv_slot

        for h in range(N_DEV - 1):
            c_send = lax.rem(my + (2 * N_DEV - 1 - h), N_DEV)
            acc = partial_chunk(c_send)
            if h > 0:
                acc = acc + comm_ref[h % 2].astype(jnp.float32)
            hop(h, to_send_slot=acc.astype(jnp.bfloat16))

        red = partial_chunk(my) + comm_ref[(N_DEV - 1) % 2].astype(jnp.float32)
        y = red * (1.0 / (1.0 + jnp.exp(-red)))
        out_ref[pl.ds(my * m_chunk, m_chunk), :] = y

        for t in range(N_DEV - 1):
            h = (N_DEV - 1) + t
            seed = y.astype(jnp.bfloat16) if t == 0 else None
            recv_slot = hop(h, to_send_slot=seed)
            c_recv = lax.rem(my + (2 * N_DEV - 1 - t), N_DEV)
            out_ref[pl.ds(c_recv * m_chunk, m_chunk), :] = (
                comm_ref[recv_slot].astype(jnp.float32)
            )

    return pl.pallas_call(
        body,
        out_shape=jax.ShapeDtypeStruct((m, n), jnp.float32),
        in_specs=[
            pl.BlockSpec(memory_space=pltpu.VMEM),
            pl.BlockSpec(memory_space=pltpu.VMEM),
        ],
        out_specs=pl.BlockSpec(memory_space=pltpu.VMEM),
        scratch_shapes=[
            pltpu.VMEM((k_per, n), jnp.bfloat16),
            pltpu.VMEM((2, m_chunk, n), jnp.bfloat16),
            pltpu.SemaphoreType.DMA((n_hops,)),
            pltpu.SemaphoreType.DMA((n_hops,)),
        ],
        compiler_params=pltpu.CompilerParams(
            collective_id=0,
            vmem_limit_bytes=120 * 1024 * 1024,
        ),
    )(x, w_mat)


# baseline (device time: 360607 ns/iter reference)
import jax
import jax.numpy as jnp
from jax import lax
from jax.experimental import pallas as pl
from jax.experimental.pallas import tpu as pltpu

N_DEV = 4


def kernel(x, w_mat):
    m, k_per = x.shape
    _, n = w_mat.shape
    m_chunk = m // N_DEV
    n_hops = 2 * (N_DEV - 1)

    def body(x_ref, w_ref, out_ref, comm_ref, stage_ref,
             send_sems, recv_sems, store_sem):
        my = lax.axis_index("i")
        left = lax.rem(my + (N_DEV - 1), N_DEV)
        right = lax.rem(my + 1, N_DEV)

        barrier_sem = pltpu.get_barrier_semaphore()
        for nbr in (left, right):
            pl.semaphore_signal(
                barrier_sem, inc=1,
                device_id=(nbr,), device_id_type=pl.DeviceIdType.MESH,
            )
        pl.semaphore_wait(barrier_sem, 2)

        def partial_chunk(c):
            xs = x_ref[pl.ds(c * m_chunk, m_chunk), :]
            return jnp.dot(xs, w_ref[...], preferred_element_type=jnp.float32)

        def store_chunk(c, val_f32):
            stage_ref[...] = val_f32
            copy = pltpu.make_async_copy(
                stage_ref, out_ref.at[pl.ds(c * m_chunk, m_chunk), :],
                store_sem,
            )
            copy.start()
            copy.wait()

        def hop(h, seed=None):
            send_slot = h % 2
            recv_slot = (h + 1) % 2
            if seed is not None:
                comm_ref[send_slot, :, :] = seed
            rdma = pltpu.make_async_remote_copy(
                src_ref=comm_ref.at[send_slot],
                dst_ref=comm_ref.at[recv_slot],
                send_sem=send_sems.at[h],
                recv_sem=recv_sems.at[h],
                device_id=(right,),
                device_id_type=pl.DeviceIdType.MESH,
            )
            rdma.start()
            rdma.wait()
            return recv_slot

        for h in range(N_DEV - 1):
            c_send = lax.rem(my + (2 * N_DEV - 1 - h), N_DEV)
            acc = partial_chunk(c_send)
            if h > 0:
                acc = acc + comm_ref[h % 2].astype(jnp.float32)
            hop(h, seed=acc.astype(jnp.bfloat16))

        red = partial_chunk(my) + comm_ref[(N_DEV - 1) % 2].astype(jnp.float32)
        y = red * (1.0 / (1.0 + jnp.exp(-red)))
        store_chunk(my, y)

        for t in range(N_DEV - 1):
            h = (N_DEV - 1) + t
            seed = y.astype(jnp.bfloat16) if t == 0 else None
            recv_slot = hop(h, seed=seed)
            c_recv = lax.rem(my + (2 * N_DEV - 1 - t), N_DEV)
            store_chunk(c_recv, comm_ref[recv_slot].astype(jnp.float32))

    x16 = x.astype(jnp.bfloat16)
    w16 = w_mat.astype(jnp.bfloat16)

    return pl.pallas_call(
        body,
        out_shape=jax.ShapeDtypeStruct((m, n), jnp.float32),
        in_specs=[
            pl.BlockSpec(memory_space=pltpu.VMEM),
            pl.BlockSpec(memory_space=pltpu.VMEM),
        ],
        out_specs=pl.BlockSpec(memory_space=pl.ANY),
        scratch_shapes=[
            pltpu.VMEM((2, m_chunk, n), jnp.bfloat16),
            pltpu.VMEM((m_chunk, n), jnp.float32),
            pltpu.SemaphoreType.DMA((n_hops,)),
            pltpu.SemaphoreType.DMA((n_hops,)),
            pltpu.SemaphoreType.DMA,
        ],
        compiler_params=pltpu.CompilerParams(
            collective_id=0,
            vmem_limit_bytes=60 * 1024 * 1024,
        ),
    )(x16, w16)


# device time: 215018 ns/iter; 1.6771x vs baseline; 1.6771x over previous
import jax
import jax.numpy as jnp
from jax import lax
from jax.experimental import pallas as pl
from jax.experimental.pallas import tpu as pltpu

N_DEV = 4


def kernel(x, w_mat):
    m, k_per = x.shape
    _, n = w_mat.shape
    m_chunk = m // N_DEV
    n_half = n // 2
    n_hops = 2 * (N_DEV - 1)

    def body(x_ref, w_ref, out_ref, comm_r_ref, comm_l_ref, stage_ref,
             send_sems_r, recv_sems_r, send_sems_l, recv_sems_l, store_sems):
        my = lax.axis_index("i")
        left = lax.rem(my + (N_DEV - 1), N_DEV)
        right = lax.rem(my + 1, N_DEV)

        barrier_sem = pltpu.get_barrier_semaphore()
        for nbr in (left, right):
            pl.semaphore_signal(
                barrier_sem, inc=1,
                device_id=(nbr,), device_id_type=pl.DeviceIdType.MESH,
            )
        pl.semaphore_wait(barrier_sem, 2)

        def partial_chunk(c, half):
            xs = x_ref[pl.ds(c * m_chunk, m_chunk), :]
            ws = w_ref[:, pl.ds(half * n_half, n_half)]
            return jnp.dot(xs, ws, preferred_element_type=jnp.float32)

        def make_hop(h):
            s, r = h % 2, (h + 1) % 2
            rd_r = pltpu.make_async_remote_copy(
                src_ref=comm_r_ref.at[s], dst_ref=comm_r_ref.at[r],
                send_sem=send_sems_r.at[h], recv_sem=recv_sems_r.at[h],
                device_id=(right,), device_id_type=pl.DeviceIdType.MESH,
            )
            rd_l = pltpu.make_async_remote_copy(
                src_ref=comm_l_ref.at[s], dst_ref=comm_l_ref.at[r],
                send_sem=send_sems_l.at[h], recv_sem=recv_sems_l.at[h],
                device_id=(left,), device_id_type=pl.DeviceIdType.MESH,
            )
            return rd_r, rd_l

        def c_right(h):
            return lax.rem(my + (2 * N_DEV - 1 - h), N_DEV)

        def c_left(h):
            return lax.rem(my + 1 + h, N_DEV)

        p_r = partial_chunk(c_right(0), 0)
        p_l = partial_chunk(c_left(0), 1)
        comm_r_ref[0, :, :] = p_r.astype(jnp.bfloat16)
        comm_l_ref[0, :, :] = p_l.astype(jnp.bfloat16)
        prev = make_hop(0)
        prev[0].start()
        prev[1].start()

        for h in range(1, N_DEV - 1):
            p_r = partial_chunk(c_right(h), 0)
            p_l = partial_chunk(c_left(h), 1)
            prev[0].wait()
            prev[1].wait()
            s = h % 2
            comm_r_ref[s, :, :] = (
                p_r + comm_r_ref[s].astype(jnp.float32)
            ).astype(jnp.bfloat16)
            comm_l_ref[s, :, :] = (
                p_l + comm_l_ref[s].astype(jnp.float32)
            ).astype(jnp.bfloat16)
            prev = make_hop(h)
            prev[0].start()
            prev[1].start()

        p_r = partial_chunk(my, 0)
        p_l = partial_chunk(my, 1)
        prev[0].wait()
        prev[1].wait()
        last = (N_DEV - 1) % 2
        red_r = p_r + comm_r_ref[last].astype(jnp.float32)
        red_l = p_l + comm_l_ref[last].astype(jnp.float32)
        y_r = red_r * (1.0 / (1.0 + jnp.exp(-red_r)))
        y_l = red_l * (1.0 / (1.0 + jnp.exp(-red_l)))

        comm_r_ref[1, :, :] = y_r.astype(jnp.bfloat16)
        comm_l_ref[1, :, :] = y_l.astype(jnp.bfloat16)

        def start_store(c, half, val_f32):
            stage_ref[half, :, :] = val_f32
            copy = pltpu.make_async_copy(
                stage_ref.at[half],
                out_ref.at[pl.ds(c * m_chunk, m_chunk),
                           pl.ds(half * n_half, n_half)],
                store_sems.at[half],
            )
            copy.start()
            return copy

        pending_stores = None
        for t in range(N_DEV - 1):
            h = (N_DEV - 1) + t
            cur = make_hop(h)
            cur[0].start()
            cur[1].start()
            if t == 0:
                st_r = start_store(my, 0, y_r)
                st_l = start_store(my, 1, y_l)
            else:
                r_prev = h % 2
                st_r = start_store(c_right(t - 1), 0,
                                   comm_r_ref[r_prev].astype(jnp.float32))
                st_l = start_store(c_left(t - 1), 1,
                                   comm_l_ref[r_prev].astype(jnp.float32))
            pending_stores = (st_r, st_l)
            cur[0].wait()
            cur[1].wait()
            pending_stores[0].wait()
            pending_stores[1].wait()

        st_r = start_store(c_right(N_DEV - 2), 0,
                           comm_r_ref[0].astype(jnp.float32))
        st_l = start_store(c_left(N_DEV - 2), 1,
                           comm_l_ref[0].astype(jnp.float32))
        st_r.wait()
        st_l.wait()

    x16 = x.astype(jnp.bfloat16)
    w16 = w_mat.astype(jnp.bfloat16)

    return pl.pallas_call(
        body,
        out_shape=jax.ShapeDtypeStruct((m, n), jnp.float32),
        in_specs=[
            pl.BlockSpec(memory_space=pltpu.VMEM),
            pl.BlockSpec(memory_space=pltpu.VMEM),
        ],
        out_specs=pl.BlockSpec(memory_space=pl.ANY),
        scratch_shapes=[
            pltpu.VMEM((2, m_chunk, n_half), jnp.bfloat16),
            pltpu.VMEM((2, m_chunk, n_half), jnp.bfloat16),
            pltpu.VMEM((2, m_chunk, n_half), jnp.float32),
            pltpu.SemaphoreType.DMA((n_hops,)),
            pltpu.SemaphoreType.DMA((n_hops,)),
            pltpu.SemaphoreType.DMA((n_hops,)),
            pltpu.SemaphoreType.DMA((n_hops,)),
            pltpu.SemaphoreType.DMA((2,)),
        ],
        compiler_params=pltpu.CompilerParams(
            collective_id=0,
            vmem_limit_bytes=60 * 1024 * 1024,
        ),
    )(x16, w16)


# device time: 184575 ns/iter; 1.9537x vs baseline; 1.1649x over previous
import jax
import jax.numpy as jnp
from jax import lax
from jax.experimental import pallas as pl
from jax.experimental.pallas import tpu as pltpu

N_DEV = 4
SUBS = 4


def kernel(x, w_mat):
    m, k_per = x.shape
    _, n = w_mat.shape
    m_chunk = m // N_DEV
    n_half = n // 2
    n_hops = 2 * (N_DEV - 1)
    m_sub = m_chunk // SUBS

    def body(x_ref, w_ref, out_ref, comm_r_ref, comm_l_ref, stage_ref,
             send_sems_r, recv_sems_r, send_sems_l, recv_sems_l, store_sems):
        my = lax.axis_index("i")
        left = lax.rem(my + (N_DEV - 1), N_DEV)
        right = lax.rem(my + 1, N_DEV)

        barrier_sem = pltpu.get_barrier_semaphore()
        for nbr in (left, right):
            pl.semaphore_signal(
                barrier_sem, inc=1,
                device_id=(nbr,), device_id_type=pl.DeviceIdType.MESH,
            )
        pl.semaphore_wait(barrier_sem, 2)

        comm = (comm_r_ref, comm_l_ref)
        sems = ((send_sems_r, recv_sems_r), (send_sems_l, recv_sems_l))
        peer = (right, left)

        def partial_chunk(c, half):
            xs = x_ref[pl.ds(c * m_chunk, m_chunk), :].astype(jnp.bfloat16)
            ws = w_ref[:, pl.ds(half * n_half, n_half)]
            return jnp.dot(xs, ws, preferred_element_type=jnp.float32)

        def sub_rdma(d, h, b):
            s, r = h % 2, (h + 1) % 2
            rows = pl.ds(b * m_sub, m_sub)
            return pltpu.make_async_remote_copy(
                src_ref=comm[d].at[s, rows, :],
                dst_ref=comm[d].at[r, rows, :],
                send_sem=sems[d][0].at[h, b],
                recv_sem=sems[d][1].at[h, b],
                device_id=(peer[d],), device_id_type=pl.DeviceIdType.MESH,
            )

        def c_ring(d, i):
            return lax.rem(my + (2 * N_DEV - 1 - i), N_DEV) if d == 0 \
                else lax.rem(my + 1 + i, N_DEV)

        def start_store(d, c):
            copy = pltpu.make_async_copy(
                stage_ref.at[d],
                out_ref.at[pl.ds(c * m_chunk, m_chunk),
                           pl.ds(d * n_half, n_half)],
                store_sems.at[d],
            )
            copy.start()
            return copy

        p = [partial_chunk(c_ring(d, 0), d) for d in (0, 1)]
        for b in range(SUBS):
            rows = pl.ds(b * m_sub, m_sub)
            for d in (0, 1):
                comm[d][0, rows, :] = (
                    p[d][b * m_sub:(b + 1) * m_sub, :].astype(jnp.bfloat16))
                sub_rdma(d, 0, b).start()

        for h in range(1, N_DEV - 1):
            p = [partial_chunk(c_ring(d, h), d) for d in (0, 1)]
            s = h % 2
            for b in range(SUBS):
                rows = pl.ds(b * m_sub, m_sub)
                for d in (0, 1):
                    sub_rdma(d, h - 1, b).wait()
                    acc = (p[d][b * m_sub:(b + 1) * m_sub, :]
                           + comm[d][s, rows, :].astype(jnp.float32))
                    comm[d][s, rows, :] = acc.astype(jnp.bfloat16)
                    sub_rdma(d, h, b).start()

        p = [partial_chunk(my, d) for d in (0, 1)]
        last = (N_DEV - 1) % 2
        for b in range(SUBS):
            rows = pl.ds(b * m_sub, m_sub)
            for d in (0, 1):
                sub_rdma(d, N_DEV - 2, b).wait()
                red = (p[d][b * m_sub:(b + 1) * m_sub, :]
                       + comm[d][last, rows, :].astype(jnp.float32))
                y = red * (1.0 / (1.0 + jnp.exp(-red)))
                comm[d][last, rows, :] = y.astype(jnp.bfloat16)
                sub_rdma(d, N_DEV - 1, b).start()
                stage_ref[d, rows, :] = y
        pending = [start_store(d, my) for d in (0, 1)]

        for t in range(1, N_DEV - 1):
            h = (N_DEV - 1) + t
            for b in range(SUBS):
                for d in (0, 1):
                    sub_rdma(d, h - 1, b).wait()
                    sub_rdma(d, h, b).start()
            for st in pending:
                st.wait()
            pending = []
            for d in (0, 1):
                stage_ref[d, :, :] = comm[d][h % 2].astype(jnp.float32)
                pending.append(start_store(d, c_ring(d, t - 1)))

        for b in range(SUBS):
            for d in (0, 1):
                sub_rdma(d, n_hops - 1, b).wait()
        for st in pending:
            st.wait()
        for d in (0, 1):
            stage_ref[d, :, :] = comm[d][0].astype(jnp.float32)
            start_store(d, c_ring(d, N_DEV - 2)).wait()

    w16 = w_mat.astype(jnp.bfloat16)

    return pl.pallas_call(
        body,
        out_shape=jax.ShapeDtypeStruct((m, n), jnp.float32),
        in_specs=[
            pl.BlockSpec(memory_space=pltpu.VMEM),
            pl.BlockSpec(memory_space=pltpu.VMEM),
        ],
        out_specs=pl.BlockSpec(memory_space=pl.ANY),
        scratch_shapes=[
            pltpu.VMEM((2, m_chunk, n_half), jnp.bfloat16),
            pltpu.VMEM((2, m_chunk, n_half), jnp.bfloat16),
            pltpu.VMEM((2, m_chunk, n_half), jnp.float32),
            pltpu.SemaphoreType.DMA((n_hops, SUBS)),
            pltpu.SemaphoreType.DMA((n_hops, SUBS)),
            pltpu.SemaphoreType.DMA((n_hops, SUBS)),
            pltpu.SemaphoreType.DMA((n_hops, SUBS)),
            pltpu.SemaphoreType.DMA((2,)),
        ],
        compiler_params=pltpu.CompilerParams(
            collective_id=0,
            vmem_limit_bytes=62 * 1024 * 1024,
        ),
    )(x, w16)


# device time: 178206 ns/iter; 2.0235x vs baseline; 1.0357x over previous
import jax
import jax.numpy as jnp
from jax import lax
from jax.experimental import pallas as pl
from jax.experimental.pallas import tpu as pltpu

N_DEV = 4
SUBS = 4


def kernel(x, w_mat):
    m, k_per = x.shape
    _, n = w_mat.shape
    m_chunk = m // N_DEV
    n_half = n // 2
    n_hops = 2 * (N_DEV - 1)
    m_sub = m_chunk // SUBS

    def body(x_ref, w_ref, out_ref, comm_r_ref, comm_l_ref, stage_ref,
             send_sems_r, recv_sems_r, send_sems_l, recv_sems_l, store_sems):
        my = lax.axis_index("i")
        left = lax.rem(my + (N_DEV - 1), N_DEV)
        right = lax.rem(my + 1, N_DEV)

        barrier_sem = pltpu.get_barrier_semaphore()
        for nbr in (left, right):
            pl.semaphore_signal(
                barrier_sem, inc=1,
                device_id=(nbr,), device_id_type=pl.DeviceIdType.MESH,
            )
        pl.semaphore_wait(barrier_sem, 2)

        comm = (comm_r_ref, comm_l_ref)
        sems = ((send_sems_r, recv_sems_r), (send_sems_l, recv_sems_l))
        peer = (right, left)

        def partial_chunk(c, half):
            xs = x_ref[pl.ds(c * m_chunk, m_chunk), :].astype(jnp.bfloat16)
            ws = w_ref[:, pl.ds(half * n_half, n_half)]
            return jnp.dot(xs, ws, preferred_element_type=jnp.float32)

        def sub_rdma(d, h, b):
            s, r = h % 2, (h + 1) % 2
            rows = pl.ds(b * m_sub, m_sub)
            return pltpu.make_async_remote_copy(
                src_ref=comm[d].at[s, rows, :],
                dst_ref=comm[d].at[r, rows, :],
                send_sem=sems[d][0].at[h, b],
                recv_sem=sems[d][1].at[h, b],
                device_id=(peer[d],), device_id_type=pl.DeviceIdType.MESH,
            )

        def c_ring(d, i):
            return lax.rem(my + (2 * N_DEV - 1 - i), N_DEV) if d == 0 \
                else lax.rem(my + 1 + i, N_DEV)

        def start_store(d, c):
            copy = pltpu.make_async_copy(
                stage_ref.at[d],
                out_ref.at[pl.ds(c * m_chunk, m_chunk),
                           pl.ds(d * n_half, n_half)],
                store_sems.at[d],
            )
            copy.start()
            return copy

        for b in range(SUBS):
            rows = pl.ds(b * m_sub, m_sub)
            for d in (0, 1):
                c = c_ring(d, 0)
                xs = x_ref[pl.ds(c * m_chunk + b * m_sub, m_sub), :].astype(
                    jnp.bfloat16)
                ws = w_ref[:, pl.ds(d * n_half, n_half)]
                p_sub = jnp.dot(xs, ws, preferred_element_type=jnp.float32)
                comm[d][0, rows, :] = p_sub.astype(jnp.bfloat16)
                sub_rdma(d, 0, b).start()

        for h in range(1, N_DEV - 1):
            p = [partial_chunk(c_ring(d, h), d) for d in (0, 1)]
            s = h % 2
            for b in range(SUBS):
                rows = pl.ds(b * m_sub, m_sub)
                for d in (0, 1):
                    sub_rdma(d, h - 1, b).wait()
                    acc = (p[d][b * m_sub:(b + 1) * m_sub, :]
                           + comm[d][s, rows, :].astype(jnp.float32))
                    comm[d][s, rows, :] = acc.astype(jnp.bfloat16)
                    sub_rdma(d, h, b).start()

        p = [partial_chunk(my, d) for d in (0, 1)]
        last = (N_DEV - 1) % 2
        for b in range(SUBS):
            rows = pl.ds(b * m_sub, m_sub)
            for d in (0, 1):
                sub_rdma(d, N_DEV - 2, b).wait()
                red = (p[d][b * m_sub:(b + 1) * m_sub, :]
                       + comm[d][last, rows, :].astype(jnp.float32))
                y = red * (1.0 / (1.0 + jnp.exp(-red)))
                comm[d][last, rows, :] = y.astype(jnp.bfloat16)
                sub_rdma(d, N_DEV - 1, b).start()
                stage_ref[d, rows, :] = y
        pending = [start_store(d, my) for d in (0, 1)]

        for t in range(1, N_DEV - 1):
            h = (N_DEV - 1) + t
            for b in range(SUBS):
                for d in (0, 1):
                    sub_rdma(d, h - 1, b).wait()
                    sub_rdma(d, h, b).start()
            for st in pending:
                st.wait()
            pending = []
            for d in (0, 1):
                stage_ref[d, :, :] = comm[d][h % 2].astype(jnp.float32)
                pending.append(start_store(d, c_ring(d, t - 1)))

        for st in pending:
            st.wait()
        tail = []
        for b in range(SUBS):
            rows = pl.ds(b * m_sub, m_sub)
            for d in (0, 1):
                sub_rdma(d, n_hops - 1, b).wait()
                stage_ref[d, rows, :] = comm[d][0, rows, :].astype(jnp.float32)
                c = c_ring(d, N_DEV - 2)
                copy = pltpu.make_async_copy(
                    stage_ref.at[d, rows, :],
                    out_ref.at[pl.ds(c * m_chunk + b * m_sub, m_sub),
                               pl.ds(d * n_half, n_half)],
                    store_sems.at[d],
                )
                copy.start()
                tail.append(copy)
        for copy in tail:
            copy.wait()

    w16 = w_mat.astype(jnp.bfloat16)

    return pl.pallas_call(
        body,
        out_shape=jax.ShapeDtypeStruct((m, n), jnp.float32),
        in_specs=[
            pl.BlockSpec(memory_space=pltpu.VMEM),
            pl.BlockSpec(memory_space=pltpu.VMEM),
        ],
        out_specs=pl.BlockSpec(memory_space=pl.ANY),
        scratch_shapes=[
            pltpu.VMEM((2, m_chunk, n_half), jnp.bfloat16),
            pltpu.VMEM((2, m_chunk, n_half), jnp.bfloat16),
            pltpu.VMEM((2, m_chunk, n_half), jnp.float32),
            pltpu.SemaphoreType.DMA((n_hops, SUBS)),
            pltpu.SemaphoreType.DMA((n_hops, SUBS)),
            pltpu.SemaphoreType.DMA((n_hops, SUBS)),
            pltpu.SemaphoreType.DMA((n_hops, SUBS)),
            pltpu.SemaphoreType.DMA((2,)),
        ],
        compiler_params=pltpu.CompilerParams(
            collective_id=0,
            vmem_limit_bytes=62 * 1024 * 1024,
        ),
    )(x, w16)


# device time: 164898 ns/iter; 2.1868x vs baseline; 1.0807x over previous
import jax
import jax.numpy as jnp
from jax import lax
from jax.experimental import pallas as pl
from jax.experimental.pallas import tpu as pltpu

N_DEV = 4
SUBS = 4


def kernel(x, w_mat):
    m, k_per = x.shape
    _, n = w_mat.shape
    m_chunk = m // N_DEV
    n_half = n // 2
    n_hops = 2 * (N_DEV - 1)
    m_sub = m_chunk // SUBS

    def body(x_ref, w_ref, out_ref, w16_ref, comm_r_ref, comm_l_ref,
             send_sems_r, recv_sems_r, send_sems_l, recv_sems_l, store_sems):
        my = lax.axis_index("i")
        left = lax.rem(my + (N_DEV - 1), N_DEV)
        right = lax.rem(my + 1, N_DEV)

        barrier_sem = pltpu.get_barrier_semaphore()
        for nbr in (left, right):
            pl.semaphore_signal(
                barrier_sem, inc=1,
                device_id=(nbr,), device_id_type=pl.DeviceIdType.MESH,
            )
        w16_ref[...] = w_ref[...].astype(jnp.bfloat16)
        pl.semaphore_wait(barrier_sem, 2)

        comm = (comm_r_ref, comm_l_ref)
        sems = ((send_sems_r, recv_sems_r), (send_sems_l, recv_sems_l))
        peer = (right, left)

        def partial_chunk(c, half):
            xs = x_ref[pl.ds(c * m_chunk, m_chunk), :].astype(jnp.bfloat16)
            ws = w16_ref[:, pl.ds(half * n_half, n_half)]
            return jnp.dot(xs, ws, preferred_element_type=jnp.float32)

        def sub_rdma(d, h, b):
            s, r = h % 2, (h + 1) % 2
            rows = pl.ds(b * m_sub, m_sub)
            return pltpu.make_async_remote_copy(
                src_ref=comm[d].at[s, rows, :],
                dst_ref=comm[d].at[r, rows, :],
                send_sem=sems[d][0].at[h, b],
                recv_sem=sems[d][1].at[h, b],
                device_id=(peer[d],), device_id_type=pl.DeviceIdType.MESH,
            )

        def c_ring(d, i):
            return lax.rem(my + (2 * N_DEV - 1 - i), N_DEV) if d == 0 \
                else lax.rem(my + 1 + i, N_DEV)

        def start_store(d, c, slot):
            copy = pltpu.make_async_copy(
                comm[d].at[slot],
                out_ref.at[pl.ds(c * m_chunk, m_chunk),
                           pl.ds(d * n_half, n_half)],
                store_sems.at[d],
            )
            copy.start()
            return copy

        for b in range(SUBS):
            rows = pl.ds(b * m_sub, m_sub)
            for d in (0, 1):
                c = c_ring(d, 0)
                xs = x_ref[pl.ds(c * m_chunk + b * m_sub, m_sub), :].astype(
                    jnp.bfloat16)
                ws = w16_ref[:, pl.ds(d * n_half, n_half)]
                p_sub = jnp.dot(xs, ws, preferred_element_type=jnp.float32)
                comm[d][0, rows, :] = p_sub.astype(jnp.bfloat16)
                sub_rdma(d, 0, b).start()

        for h in range(1, N_DEV - 1):
            p = [partial_chunk(c_ring(d, h), d) for d in (0, 1)]
            s = h % 2
            for b in range(SUBS):
                rows = pl.ds(b * m_sub, m_sub)
                for d in (0, 1):
                    sub_rdma(d, h - 1, b).wait()
                    acc = (p[d][b * m_sub:(b + 1) * m_sub, :]
                           + comm[d][s, rows, :].astype(jnp.float32))
                    comm[d][s, rows, :] = acc.astype(jnp.bfloat16)
                    sub_rdma(d, h, b).start()

        p = [partial_chunk(my, d) for d in (0, 1)]
        last = (N_DEV - 1) % 2
        for b in range(SUBS):
            rows = pl.ds(b * m_sub, m_sub)
            for d in (0, 1):
                sub_rdma(d, N_DEV - 2, b).wait()
                red = (p[d][b * m_sub:(b + 1) * m_sub, :]
                       + comm[d][last, rows, :].astype(jnp.float32))
                y = red * (1.0 / (1.0 + jnp.exp(-red)))
                comm[d][last, rows, :] = y.astype(jnp.bfloat16)
                sub_rdma(d, N_DEV - 1, b).start()
        pending = [start_store(d, my, last) for d in (0, 1)]

        for t in range(1, N_DEV - 1):
            h = (N_DEV - 1) + t
            for b in range(SUBS):
                for d in (0, 1):
                    sub_rdma(d, h - 1, b).wait()
                    sub_rdma(d, h, b).start()
            for st in pending:
                st.wait()
            pending = [start_store(d, c_ring(d, t - 1), h % 2)
                       for d in (0, 1)]

        for st in pending:
            st.wait()
        tail = []
        for b in range(SUBS):
            rows = pl.ds(b * m_sub, m_sub)
            for d in (0, 1):
                sub_rdma(d, n_hops - 1, b).wait()
                c = c_ring(d, N_DEV - 2)
                copy = pltpu.make_async_copy(
                    comm[d].at[0, rows, :],
                    out_ref.at[pl.ds(c * m_chunk + b * m_sub, m_sub),
                               pl.ds(d * n_half, n_half)],
                    store_sems.at[d],
                )
                copy.start()
                tail.append(copy)
        for copy in tail:
            copy.wait()

    return pl.pallas_call(
        body,
        out_shape=jax.ShapeDtypeStruct((m, n), jnp.bfloat16),
        in_specs=[
            pl.BlockSpec(memory_space=pltpu.VMEM),
            pl.BlockSpec(memory_space=pltpu.VMEM),
        ],
        out_specs=pl.BlockSpec(memory_space=pl.ANY),
        scratch_shapes=[
            pltpu.VMEM((k_per, n), jnp.bfloat16),
            pltpu.VMEM((2, m_chunk, n_half), jnp.bfloat16),
            pltpu.VMEM((2, m_chunk, n_half), jnp.bfloat16),
            pltpu.SemaphoreType.DMA((n_hops, SUBS)),
            pltpu.SemaphoreType.DMA((n_hops, SUBS)),
            pltpu.SemaphoreType.DMA((n_hops, SUBS)),
            pltpu.SemaphoreType.DMA((n_hops, SUBS)),
            pltpu.SemaphoreType.DMA((2,)),
        ],
        compiler_params=pltpu.CompilerParams(
            collective_id=0,
            vmem_limit_bytes=62 * 1024 * 1024,
        ),
    )(x, w_mat)
